# baseline (speedup 1.0000x reference)
import functools

import jax
import jax.numpy as jnp
from jax import lax
from jax.experimental import pallas as pl
from jax.experimental.pallas import tpu as pltpu
from jax.experimental.pallas import tpu_sc as plsc

_L = 16


def _round_half_even_div(v, s):
    q = v.astype(jnp.float32) / s
    t = q.astype(jnp.int32)
    frac = q - t.astype(jnp.float32)
    odd = (t & 1) == 1
    up = (frac > 0.5) | ((frac == 0.5) & odd)
    return t + jnp.where(up, 1, 0).astype(jnp.int32)


def kernel(image_features, c2l_proj_ind, scale):
    B, C, H, W = image_features.shape
    HW = H * W
    _, _, Hrv, Wrv = c2l_proj_ind.shape
    NPIX = Hrv * Wrv

    NC, NS = 2, 16
    NW = NC * NS
    WPB = NW // B
    CPW = C // WPB

    CHUNK = 4096
    NCH1 = NPIX // CHUNK
    OUTC = 8192
    NOC = NPIX // OUTC
    U1 = 4
    U2 = 8
    PAD = _L

    feat1 = image_features.reshape(B * C * HW)
    proj1 = c2l_proj_ind.reshape(B * 2 * NPIX)
    scale_b = jnp.broadcast_to(
        scale.astype(jnp.float32).reshape(2, 1), (2, _L))

    mesh = plsc.VectorSubcoreMesh(core_axis_name="c", subcore_axis_name="s")

    @functools.partial(
        pl.kernel,
        mesh=mesh,
        compiler_params=pltpu.CompilerParams(needs_layout_passes=False),
        out_type=jax.ShapeDtypeStruct((B * C * NPIX,), jnp.float32),
        scratch_types=[
            pltpu.VMEM((HW + PAD,), jnp.float32),
            pltpu.VMEM((HW + PAD,), jnp.float32),
            pltpu.VMEM((NPIX // 2,), jnp.int32),
            pltpu.VMEM((CHUNK,), jnp.int32),
            pltpu.VMEM((CHUNK,), jnp.int32),
            pltpu.VMEM((CHUNK,), jnp.int32),
            pltpu.VMEM((CHUNK,), jnp.int32),
            pltpu.VMEM((OUTC,), jnp.float32),
            pltpu.VMEM((OUTC,), jnp.float32),
            pltpu.VMEM((2, _L), jnp.float32),
            pltpu.VMEM_SHARED((2, NPIX // 2), jnp.int32),
            pltpu.SemaphoreType.DMA,
            pltpu.SemaphoreType.DMA,
            pltpu.SemaphoreType.DMA,
            pltpu.SemaphoreType.DMA,
            pltpu.SemaphoreType.DMA,
            pltpu.SemaphoreType.DMA,
        ],
    )
    def _sc_kernel(feat_hbm, proj_hbm, scale_hbm, out_hbm,
                   plane_a, plane_b, flatw_v,
                   stgy_a, stgy_b, stgx_a, stgx_b,
                   out_a, out_b, scale_v, flatw_sh,
                   sem_sa, sem_sb, sem_pa, sem_pb, sem_oa, sem_ob):
        cid = lax.axis_index("c")
        sid = lax.axis_index("s")
        w = cid * NS + sid
        b = w // WPB
        cg = w % WPB
        gb = sid // WPB
        wg = sid % WPB

        planes = (plane_a, plane_b)
        stgy = (stgy_a, stgy_b)
        stgx = (stgx_a, stgx_b)
        outb = (out_a, out_b)
        sem_s = (sem_sa, sem_sb)
        sem_p = (sem_pa, sem_pb)
        sem_o = (sem_oa, sem_ob)

        pltpu.sync_copy(scale_hbm, scale_v)
        s0 = scale_v[0, :]
        s1 = scale_v[1, :]

        plane_a[pl.ds(HW, PAD)] = jnp.zeros((PAD,), jnp.float32)
        plane_b[pl.ds(HW, PAD)] = jnp.zeros((PAD,), jnp.float32)

        def _flat_idx(yv, xv):
            ry = _round_half_even_div(yv, s0)
            rx = _round_half_even_div(xv, s1)
            mask = (ry < H) & (rx < W)
            return jnp.where(mask, ry * W + rx, HW)

        def plane_start(ci, p):
            ch = cg * CPW + ci
            return pltpu.async_copy(
                feat_hbm.at[pl.ds((b * C + ch) * HW, HW)],
                planes[p].at[pl.ds(0, HW)], sem_p[p])

        plane_pend = plane_start(0, 0)

        PXW = NPIX // WPB
        NCW1 = PXW // CHUNK
        pxbase = wg * PXW

        def stage_start(ci, p):
            off = pxbase + ci * CHUNK
            cy = pltpu.async_copy(
                proj_hbm.at[pl.ds(b * 2 * NPIX + off, CHUNK)],
                stgy[p], sem_s[p])
            cx = pltpu.async_copy(
                proj_hbm.at[pl.ds((b * 2 + 1) * NPIX + off, CHUNK)],
                stgx[p], sem_s[p])
            return (cy, cx)

        pend = stage_start(0, 0)
        for ci in range(NCW1):
            p = ci & 1
            pend[0].wait()
            pend[1].wait()
            if ci + 1 < NCW1:
                pend = stage_start(ci + 1, (ci + 1) & 1)
            woff = wg * (PXW // 2) + ci * (CHUNK // 2)

            @plsc.parallel_loop(0, CHUNK // 32, unroll=U1)
            def p1_grp(j, _p=p, _woff=woff):
                o = j * 32
                f0 = _flat_idx(stgy[_p][pl.ds(o, _L)],
                               stgx[_p][pl.ds(o, _L)])
                f1 = _flat_idx(stgy[_p][pl.ds(o + _L, _L)],
                               stgx[_p][pl.ds(o + _L, _L)])
                flatw_v[pl.ds(_woff + j * _L, _L)] = f0 | (f1 << 16)

        pltpu.sync_copy(flatw_v.at[pl.ds(wg * (PXW // 2), PXW // 2)],
                        flatw_sh.at[gb, pl.ds(wg * (PXW // 2), PXW // 2)])
        plsc.subcore_barrier()
        pltpu.sync_copy(flatw_sh.at[gb], flatw_v)

        out_pend = [None, None]
        for ci in range(CPW):
            pp = ci & 1
            ch = cg * CPW + ci
            plane_pend.wait()
            if ci + 1 < CPW:
                plane_pend = plane_start(ci + 1, (ci + 1) & 1)

            for oi in range(NOC):
                op = oi & 1
                if out_pend[op] is not None:
                    out_pend[op].wait()

                @plsc.parallel_loop(0, OUTC // 32, unroll=U2)
                def grp(j, _pp=pp, _op=op, _oi=oi):
                    word = flatw_v[pl.ds(_oi * (OUTC // 2) + j * _L, _L)]
                    idx0 = word & 0xFFFF
                    idx1 = lax.shift_right_logical(word, 16)
                    outb[_op][pl.ds(j * 32, _L)] = (
                        plsc.load_gather(planes[_pp], [idx0]))
                    outb[_op][pl.ds(j * 32 + _L, _L)] = (
                        plsc.load_gather(planes[_pp], [idx1]))
                out_pend[op] = pltpu.async_copy(
                    outb[op],
                    out_hbm.at[pl.ds((b * C + ch) * NPIX + oi * OUTC, OUTC)],
                    sem_o[op])

        for op in range(2):
            if out_pend[op] is not None:
                out_pend[op].wait()

    out = _sc_kernel(feat1, proj1, scale_b)
    return out.reshape(B, C, Hrv, Wrv)


# speedup vs baseline: 1.0240x; 1.0240x over previous
import functools

import jax
import jax.numpy as jnp
from jax import lax
from jax.experimental import pallas as pl
from jax.experimental.pallas import tpu as pltpu
from jax.experimental.pallas import tpu_sc as plsc

_L = 16


def _round_half_even_div(v, s):
    q = v.astype(jnp.float32) / s
    t = q.astype(jnp.int32)
    frac = q - t.astype(jnp.float32)
    odd = (t & 1) == 1
    up = (frac > 0.5) | ((frac == 0.5) & odd)
    return t + jnp.where(up, 1, 0).astype(jnp.int32)


def kernel(image_features, c2l_proj_ind, scale):
    B, C, H, W = image_features.shape
    HW = H * W
    _, _, Hrv, Wrv = c2l_proj_ind.shape
    NPIX = Hrv * Wrv

    NC, NS = 2, 16
    NW = NC * NS
    WPB = NW // B
    CPW = C // WPB

    CHUNK = 4096
    NCH1 = NPIX // CHUNK
    OUTC = 8192
    NOC = NPIX // OUTC
    U1 = 4
    U2 = 8
    PAD = _L

    feat3 = image_features.reshape(B, C, HW)
    proj3 = c2l_proj_ind.reshape(B, 2, NPIX)
    scale_b = jnp.broadcast_to(
        scale.astype(jnp.float32).reshape(2, 1), (2, _L))

    mesh = plsc.VectorSubcoreMesh(core_axis_name="c", subcore_axis_name="s")

    @functools.partial(
        pl.kernel,
        mesh=mesh,
        compiler_params=pltpu.CompilerParams(needs_layout_passes=False),
        out_type=jax.ShapeDtypeStruct((B * C * NPIX,), jnp.float32),
        scratch_types=[
            pltpu.VMEM((HW + PAD,), jnp.float32),
            pltpu.VMEM((HW + PAD,), jnp.float32),
            pltpu.VMEM((NPIX // 2,), jnp.int32),
            pltpu.VMEM((CHUNK,), jnp.int32),
            pltpu.VMEM((CHUNK,), jnp.int32),
            pltpu.VMEM((CHUNK,), jnp.int32),
            pltpu.VMEM((CHUNK,), jnp.int32),
            pltpu.VMEM((OUTC,), jnp.float32),
            pltpu.VMEM((OUTC,), jnp.float32),
            pltpu.VMEM((2, _L), jnp.float32),
            pltpu.VMEM_SHARED((2, NPIX // 2), jnp.int32),
            pltpu.SemaphoreType.DMA,
            pltpu.SemaphoreType.DMA,
            pltpu.SemaphoreType.DMA,
            pltpu.SemaphoreType.DMA,
            pltpu.SemaphoreType.DMA,
            pltpu.SemaphoreType.DMA,
        ],
    )
    def _sc_kernel(feat_hbm, proj_hbm, scale_hbm, out_hbm,
                   plane_a, plane_b, flatw_v,
                   stgy_a, stgy_b, stgx_a, stgx_b,
                   out_a, out_b, scale_v, flatw_sh,
                   sem_sa, sem_sb, sem_pa, sem_pb, sem_oa, sem_ob):
        cid = lax.axis_index("c")
        sid = lax.axis_index("s")
        w = cid * NS + sid
        b = w // WPB
        cg = w % WPB
        gb = sid // WPB
        wg = sid % WPB

        planes = (plane_a, plane_b)
        stgy = (stgy_a, stgy_b)
        stgx = (stgx_a, stgx_b)
        outb = (out_a, out_b)
        sem_s = (sem_sa, sem_sb)
        sem_p = (sem_pa, sem_pb)
        sem_o = (sem_oa, sem_ob)

        pltpu.sync_copy(scale_hbm, scale_v)
        s0 = scale_v[0, :]
        s1 = scale_v[1, :]

        plane_a[pl.ds(HW, PAD)] = jnp.zeros((PAD,), jnp.float32)
        plane_b[pl.ds(HW, PAD)] = jnp.zeros((PAD,), jnp.float32)

        def _flat_idx(yv, xv):
            ry = _round_half_even_div(yv, s0)
            rx = _round_half_even_div(xv, s1)
            mask = (ry < H) & (rx < W)
            return jnp.where(mask, ry * W + rx, HW)

        def plane_start(ci, p):
            ch = cg * CPW + ci
            return pltpu.async_copy(
                feat_hbm.at[b, ch], planes[p].at[pl.ds(0, HW)], sem_p[p])

        plane_pend = plane_start(0, 0)

        PXW = NPIX // WPB
        NCW1 = PXW // CHUNK
        pxbase = wg * PXW

        def stage_start(ci, p):
            off = pxbase + ci * CHUNK
            cy = pltpu.async_copy(
                proj_hbm.at[b, 0, pl.ds(off, CHUNK)], stgy[p], sem_s[p])
            cx = pltpu.async_copy(
                proj_hbm.at[b, 1, pl.ds(off, CHUNK)], stgx[p], sem_s[p])
            return (cy, cx)

        pend = stage_start(0, 0)
        for ci in range(NCW1):
            p = ci & 1
            pend[0].wait()
            pend[1].wait()
            if ci + 1 < NCW1:
                pend = stage_start(ci + 1, (ci + 1) & 1)
            woff = wg * (PXW // 2) + ci * (CHUNK // 2)

            @plsc.parallel_loop(0, CHUNK // 32, unroll=U1)
            def p1_grp(j, _p=p, _woff=woff):
                o = j * 32
                f0 = _flat_idx(stgy[_p][pl.ds(o, _L)],
                               stgx[_p][pl.ds(o, _L)])
                f1 = _flat_idx(stgy[_p][pl.ds(o + _L, _L)],
                               stgx[_p][pl.ds(o + _L, _L)])
                flatw_v[pl.ds(_woff + j * _L, _L)] = f0 | (f1 << 16)

        pltpu.sync_copy(flatw_v.at[pl.ds(wg * (PXW // 2), PXW // 2)],
                        flatw_sh.at[gb, pl.ds(wg * (PXW // 2), PXW // 2)])
        plsc.subcore_barrier()
        pltpu.sync_copy(flatw_sh.at[gb], flatw_v)

        out_pend = [None, None]
        for ci in range(CPW):
            pp = ci & 1
            ch = cg * CPW + ci
            plane_pend.wait()
            if ci + 1 < CPW:
                plane_pend = plane_start(ci + 1, (ci + 1) & 1)

            for oi in range(NOC):
                op = oi & 1
                if out_pend[op] is not None:
                    out_pend[op].wait()

                @plsc.parallel_loop(0, OUTC // 32, unroll=U2)
                def grp(j, _pp=pp, _op=op, _oi=oi):
                    word = flatw_v[pl.ds(_oi * (OUTC // 2) + j * _L, _L)]
                    idx0 = word & 0xFFFF
                    idx1 = lax.shift_right_logical(word, 16)
                    outb[_op][pl.ds(j * 32, _L)] = (
                        plsc.load_gather(planes[_pp], [idx0]))
                    outb[_op][pl.ds(j * 32 + _L, _L)] = (
                        plsc.load_gather(planes[_pp], [idx1]))
                out_pend[op] = pltpu.async_copy(
                    outb[op],
                    out_hbm.at[pl.ds((b * C + ch) * NPIX + oi * OUTC, OUTC)],
                    sem_o[op])

        for op in range(2):
            if out_pend[op] is not None:
                out_pend[op].wait()

    out = _sc_kernel(feat3, proj3, scale_b)
    return out.reshape(B, C, Hrv, Wrv)


# speedup vs baseline: 1.4941x; 1.4591x over previous
import functools

import jax
import jax.numpy as jnp
from jax import lax
from jax.experimental import pallas as pl
from jax.experimental.pallas import tpu as pltpu
from jax.experimental.pallas import tpu_sc as plsc

_L = 16


def _round_half_even_div(v, s):
    q = v.astype(jnp.float32) / s
    t = q.astype(jnp.int32)
    frac = q - t.astype(jnp.float32)
    odd = (t & 1) == 1
    up = (frac > 0.5) | ((frac == 0.5) & odd)
    return t + jnp.where(up, 1, 0).astype(jnp.int32)


def kernel(image_features, c2l_proj_ind, scale):
    B, C, H, W = image_features.shape
    HW = H * W
    _, _, Hrv, Wrv = c2l_proj_ind.shape
    NPIX = Hrv * Wrv

    NC, NS = 2, 16
    NW = NC * NS
    WPB = NW // B
    CPW = C // WPB

    CHUNK = 4096
    NCH1 = NPIX // CHUNK
    OUTC = 8192
    NOC = NPIX // OUTC
    U1 = 4
    U2 = 8
    PAD = _L

    feat3 = image_features.reshape(B, C, HW)
    proj3 = c2l_proj_ind.reshape(B, 2, NPIX)
    scale_b = jnp.broadcast_to(
        scale.astype(jnp.float32).reshape(2, 1), (2, _L))

    mesh = plsc.VectorSubcoreMesh(core_axis_name="c", subcore_axis_name="s")

    @functools.partial(
        pl.kernel,
        mesh=mesh,
        compiler_params=pltpu.CompilerParams(needs_layout_passes=False),
        out_type=jax.ShapeDtypeStruct((B, C, Hrv, Wrv), jnp.float32),
        scratch_types=[
            pltpu.VMEM((HW + PAD,), jnp.float32),
            pltpu.VMEM((HW + PAD,), jnp.float32),
            pltpu.VMEM((NPIX // 2,), jnp.int32),
            pltpu.VMEM((CHUNK,), jnp.int32),
            pltpu.VMEM((CHUNK,), jnp.int32),
            pltpu.VMEM((CHUNK,), jnp.int32),
            pltpu.VMEM((CHUNK,), jnp.int32),
            pltpu.VMEM((OUTC // Wrv, Wrv), jnp.float32),
            pltpu.VMEM((OUTC // Wrv, Wrv), jnp.float32),
            pltpu.VMEM((2, _L), jnp.float32),
            pltpu.VMEM_SHARED((2, NPIX // 2), jnp.int32),
            pltpu.SemaphoreType.DMA,
            pltpu.SemaphoreType.DMA,
            pltpu.SemaphoreType.DMA,
            pltpu.SemaphoreType.DMA,
            pltpu.SemaphoreType.DMA,
            pltpu.SemaphoreType.DMA,
        ],
    )
    def _sc_kernel(feat_hbm, proj_hbm, scale_hbm, out_hbm,
                   plane_a, plane_b, flatw_v,
                   stgy_a, stgy_b, stgx_a, stgx_b,
                   out_a, out_b, scale_v, flatw_sh,
                   sem_sa, sem_sb, sem_pa, sem_pb, sem_oa, sem_ob):
        cid = lax.axis_index("c")
        sid = lax.axis_index("s")
        w = cid * NS + sid
        b = w // WPB
        cg = w % WPB
        gb = sid // WPB
        wg = sid % WPB

        planes = (plane_a, plane_b)
        stgy = (stgy_a, stgy_b)
        stgx = (stgx_a, stgx_b)
        outb = (out_a, out_b)
        sem_s = (sem_sa, sem_sb)
        sem_p = (sem_pa, sem_pb)
        sem_o = (sem_oa, sem_ob)

        pltpu.sync_copy(scale_hbm, scale_v)
        s0 = scale_v[0, :]
        s1 = scale_v[1, :]

        plane_a[pl.ds(HW, PAD)] = jnp.zeros((PAD,), jnp.float32)
        plane_b[pl.ds(HW, PAD)] = jnp.zeros((PAD,), jnp.float32)

        def _flat_idx(yv, xv):
            ry = _round_half_even_div(yv, s0)
            rx = _round_half_even_div(xv, s1)
            mask = (ry < H) & (rx < W)
            return jnp.where(mask, ry * W + rx, HW)

        def plane_start(ci, p):
            ch = cg * CPW + ci
            return pltpu.async_copy(
                feat_hbm.at[b, ch], planes[p].at[pl.ds(0, HW)], sem_p[p])

        plane_pend = plane_start(0, 0)

        PXW = NPIX // WPB
        NCW1 = PXW // CHUNK
        pxbase = wg * PXW

        def stage_start(ci, p):
            off = pxbase + ci * CHUNK
            cy = pltpu.async_copy(
                proj_hbm.at[b, 0, pl.ds(off, CHUNK)], stgy[p], sem_s[p])
            cx = pltpu.async_copy(
                proj_hbm.at[b, 1, pl.ds(off, CHUNK)], stgx[p], sem_s[p])
            return (cy, cx)

        pend = stage_start(0, 0)
        for ci in range(NCW1):
            p = ci & 1
            pend[0].wait()
            pend[1].wait()
            if ci + 1 < NCW1:
                pend = stage_start(ci + 1, (ci + 1) & 1)
            woff = wg * (PXW // 2) + ci * (CHUNK // 2)

            @plsc.parallel_loop(0, CHUNK // 32, unroll=U1)
            def p1_grp(j, _p=p, _woff=woff):
                o = j * 32
                f0 = _flat_idx(stgy[_p][pl.ds(o, _L)],
                               stgx[_p][pl.ds(o, _L)])
                f1 = _flat_idx(stgy[_p][pl.ds(o + _L, _L)],
                               stgx[_p][pl.ds(o + _L, _L)])
                flatw_v[pl.ds(_woff + j * _L, _L)] = f0 | (f1 << 16)

        pltpu.sync_copy(flatw_v.at[pl.ds(wg * (PXW // 2), PXW // 2)],
                        flatw_sh.at[gb, pl.ds(wg * (PXW // 2), PXW // 2)])
        plsc.subcore_barrier()
        pltpu.sync_copy(flatw_sh.at[gb], flatw_v)

        out_pend = [None, None]
        for ci in range(CPW):
            pp = ci & 1
            ch = cg * CPW + ci
            plane_pend.wait()
            if ci + 1 < CPW:
                plane_pend = plane_start(ci + 1, (ci + 1) & 1)

            for oi in range(NOC):
                op = oi & 1
                if out_pend[op] is not None:
                    out_pend[op].wait()

                GPR = Wrv // 32

                @plsc.parallel_loop(0, OUTC // 32, unroll=U2)
                def grp(j, _pp=pp, _op=op, _oi=oi):
                    word = flatw_v[pl.ds(_oi * (OUTC // 2) + j * _L, _L)]
                    idx0 = word & 0xFFFF
                    idx1 = lax.shift_right_logical(word, 16)
                    r = j // GPR
                    col = (j % GPR) * 32
                    outb[_op][r, pl.ds(col, _L)] = (
                        plsc.load_gather(planes[_pp], [idx0]))
                    outb[_op][r, pl.ds(col + _L, _L)] = (
                        plsc.load_gather(planes[_pp], [idx1]))
                out_pend[op] = pltpu.async_copy(
                    outb[op],
                    out_hbm.at[b, ch,
                               pl.ds(oi * (OUTC // Wrv), OUTC // Wrv), :],
                    sem_o[op])

        for op in range(2):
            if out_pend[op] is not None:
                out_pend[op].wait()

    return _sc_kernel(feat3, proj3, scale_b)


# speedup vs baseline: 1.5482x; 1.0362x over previous
import functools

import jax
import jax.numpy as jnp
from jax import lax
from jax.experimental import pallas as pl
from jax.experimental.pallas import tpu as pltpu
from jax.experimental.pallas import tpu_sc as plsc

_L = 16


def _round_half_even_div(v, s):
    q = v.astype(jnp.float32) / s
    t = q.astype(jnp.int32)
    frac = q - t.astype(jnp.float32)
    odd = (t & 1) == 1
    up = (frac > 0.5) | ((frac == 0.5) & odd)
    return t + jnp.where(up, 1, 0).astype(jnp.int32)


def kernel(image_features, c2l_proj_ind, scale):
    B, C, H, W = image_features.shape
    HW = H * W
    _, _, Hrv, Wrv = c2l_proj_ind.shape
    NPIX = Hrv * Wrv

    NC, NS = 2, 16
    NW = NC * NS
    WPB = NW // B
    CPW = C // WPB

    CHUNK = 8192
    OUTC = 8192
    NOC = NPIX // OUTC
    U1 = 4
    U2 = 8
    PAD = _L

    feat3 = image_features.reshape(B, C, HW)
    scale_b = jnp.broadcast_to(
        scale.astype(jnp.float32).reshape(2, 1), (2, _L))

    mesh = plsc.VectorSubcoreMesh(core_axis_name="c", subcore_axis_name="s")

    @functools.partial(
        pl.kernel,
        mesh=mesh,
        compiler_params=pltpu.CompilerParams(needs_layout_passes=False),
        out_type=jax.ShapeDtypeStruct((B, C, Hrv, Wrv), jnp.float32),
        scratch_types=[
            pltpu.VMEM((HW + PAD,), jnp.float32),
            pltpu.VMEM((HW + PAD,), jnp.float32),
            pltpu.VMEM((NPIX // 2,), jnp.int32),
            pltpu.VMEM((CHUNK // Wrv, Wrv), jnp.int32),
            pltpu.VMEM((CHUNK // Wrv, Wrv), jnp.int32),
            pltpu.VMEM((OUTC // Wrv, Wrv), jnp.float32),
            pltpu.VMEM((OUTC // Wrv, Wrv), jnp.float32),
            pltpu.VMEM((2, _L), jnp.float32),
            pltpu.VMEM_SHARED((2, NPIX // 2), jnp.int32),
            pltpu.SemaphoreType.DMA,
            pltpu.SemaphoreType.DMA,
            pltpu.SemaphoreType.DMA,
            pltpu.SemaphoreType.DMA,
            pltpu.SemaphoreType.DMA,
        ],
    )
    def _sc_kernel(feat_hbm, proj_hbm, scale_hbm, out_hbm,
                   plane_a, plane_b, flatw_v,
                   stgy, stgx,
                   out_a, out_b, scale_v, flatw_sh,
                   sem_s, sem_pa, sem_pb, sem_oa, sem_ob):
        cid = lax.axis_index("c")
        sid = lax.axis_index("s")
        w = cid * NS + sid
        b = w // WPB
        cg = w % WPB
        gb = sid // WPB
        wg = sid % WPB

        planes = (plane_a, plane_b)
        outb = (out_a, out_b)
        sem_p = (sem_pa, sem_pb)
        sem_o = (sem_oa, sem_ob)

        pltpu.sync_copy(scale_hbm, scale_v)
        s0 = scale_v[0, :]
        s1 = scale_v[1, :]

        plane_a[pl.ds(HW, PAD)] = jnp.zeros((PAD,), jnp.float32)
        plane_b[pl.ds(HW, PAD)] = jnp.zeros((PAD,), jnp.float32)

        def _flat_idx(yv, xv):
            ry = _round_half_even_div(yv, s0)
            rx = _round_half_even_div(xv, s1)
            mask = (ry < H) & (rx < W)
            return jnp.where(mask, ry * W + rx, HW)

        def plane_start(ci, p):
            ch = cg * CPW + ci
            return pltpu.async_copy(
                feat_hbm.at[b, ch], planes[p].at[pl.ds(0, HW)], sem_p[p])

        plane_pend = plane_start(0, 0)

        PXW = NPIX // WPB
        nrows = CHUNK // Wrv
        row0 = wg * nrows
        cy = pltpu.async_copy(
            proj_hbm.at[b, 0, pl.ds(row0, nrows), :], stgy, sem_s)
        cx = pltpu.async_copy(
            proj_hbm.at[b, 1, pl.ds(row0, nrows), :], stgx, sem_s)
        cy.wait()
        cx.wait()
        woff = wg * (PXW // 2)
        SGPR = Wrv // 32

        @plsc.parallel_loop(0, CHUNK // 32, unroll=U1)
        def p1_grp(j):
            r = j // SGPR
            col = (j % SGPR) * 32
            f0 = _flat_idx(stgy[r, pl.ds(col, _L)],
                           stgx[r, pl.ds(col, _L)])
            f1 = _flat_idx(stgy[r, pl.ds(col + _L, _L)],
                           stgx[r, pl.ds(col + _L, _L)])
            flatw_v[pl.ds(woff + j * _L, _L)] = f0 | (f1 << 16)

        pltpu.sync_copy(flatw_v.at[pl.ds(wg * (PXW // 2), PXW // 2)],
                        flatw_sh.at[gb, pl.ds(wg * (PXW // 2), PXW // 2)])
        plsc.subcore_barrier()
        pltpu.sync_copy(flatw_sh.at[gb], flatw_v)

        out_pend = [None, None]
        for ci in range(CPW):
            pp = ci & 1
            ch = cg * CPW + ci
            plane_pend.wait()
            if ci + 1 < CPW:
                plane_pend = plane_start(ci + 1, (ci + 1) & 1)

            for oi in range(NOC):
                op = oi & 1
                if out_pend[op] is not None:
                    out_pend[op].wait()

                GPR = Wrv // 32

                @plsc.parallel_loop(0, OUTC // 32, unroll=U2)
                def grp(j, _pp=pp, _op=op, _oi=oi):
                    word = flatw_v[pl.ds(_oi * (OUTC // 2) + j * _L, _L)]
                    idx0 = word & 0xFFFF
                    idx1 = lax.shift_right_logical(word, 16)
                    r = j // GPR
                    col = (j % GPR) * 32
                    outb[_op][r, pl.ds(col, _L)] = (
                        plsc.load_gather(planes[_pp], [idx0]))
                    outb[_op][r, pl.ds(col + _L, _L)] = (
                        plsc.load_gather(planes[_pp], [idx1]))
                out_pend[op] = pltpu.async_copy(
                    outb[op],
                    out_hbm.at[b, ch,
                               pl.ds(oi * (OUTC // Wrv), OUTC // Wrv), :],
                    sem_o[op])

        for op in range(2):
            if out_pend[op] is not None:
                out_pend[op].wait()

    return _sc_kernel(feat3, c2l_proj_ind, scale_b)
